# baseline (speedup 1.0000x reference)
import functools

import jax
import jax.numpy as jnp
from jax import lax
from jax.experimental import pallas as pl
from jax.experimental.pallas import tpu as pltpu
from jax.experimental.pallas import tpu_sc as plsc

_B = 16384
_F = 26
_FH = 13
_V = 100000
_D = 50
_DP = 128
_NUM = 13
_BFH = _B * _FH

_NC = 2
_NS = 16
_NW = _NC * _NS

_ROWS_PER_W = _BFH // _NW
_GROUP = 128
_GROUPS_PER_CHUNK = 4
_CHUNK = _GROUP * _GROUPS_PER_CHUNK
_CHUNKS_PER_W = _ROWS_PER_W // _CHUNK

_VCHUNKS = [(k * 6400, 6400) for k in range(15)] + [(96000, 4000)]
_VCMAX = 6400


def _in_copy(t_ref, xin_v, in_sems, f, b):
    return pltpu.make_async_copy(t_ref.at[f], xin_v.at[b], in_sems.at[b])


def _pad_body(t_ref, out_ref, xin_v, xt_v, in_sems, out_sems):
    f = pl.program_id(0)
    nf = pl.num_programs(0)

    @pl.when(f == 0)
    def _():
        _in_copy(t_ref, xin_v, in_sems, 0, 0).start()

    @pl.when(f + 1 < nf)
    def _():
        _in_copy(t_ref, xin_v, in_sems, f + 1, (f + 1) % 2).start()

    _in_copy(t_ref, xin_v, in_sems, f, f % 2).wait()

    x = xin_v.at[f % 2]
    for j, (v0, vn) in enumerate(_VCHUNKS):
        xc = x[:, pl.ds(v0, vn)]
        xt = jnp.swapaxes(xc, 0, 1)
        z = jnp.zeros((vn, _DP - _D), dtype=jnp.float32)
        if j >= 2:
            v0p, vnp = _VCHUNKS[j - 2]
            pltpu.make_async_copy(
                xt_v.at[j % 2, pl.ds(0, vnp)],
                out_ref.at[pl.ds(f * _V + v0p, vnp)], out_sems.at[j % 2]).wait()
        xt_v[j % 2, pl.ds(0, vn)] = jnp.concatenate([xt, z], axis=1)
        pltpu.make_async_copy(
            xt_v.at[j % 2, pl.ds(0, vn)],
            out_ref.at[pl.ds(f * _V + v0, vn)], out_sems.at[j % 2]).start()
    for j in (len(_VCHUNKS) - 2, len(_VCHUNKS) - 1):
        v0, vn = _VCHUNKS[j]
        pltpu.make_async_copy(
            xt_v.at[j % 2, pl.ds(0, vn)],
            out_ref.at[pl.ds(f * _V + v0, vn)], out_sems.at[j % 2]).wait()


def _tc_pad(tables_t_half):
    return pl.pallas_call(
        _pad_body,
        grid=(_FH,),
        in_specs=[pl.BlockSpec(memory_space=pltpu.HBM)],
        out_specs=pl.BlockSpec(memory_space=pltpu.HBM),
        out_shape=jax.ShapeDtypeStruct((_FH * _V, _DP), jnp.float32),
        scratch_shapes=[
            pltpu.VMEM((2, _D, _V), jnp.float32),
            pltpu.VMEM((2, _VCMAX, _DP), jnp.float32),
            pltpu.SemaphoreType.DMA((2,)),
            pltpu.SemaphoreType.DMA((2,)),
        ],
    )(tables_t_half)


def _sc_gather_body(table_hbm, idx_hbm, out_hbm, idx_v, rows_v, sem):
    wid = lax.axis_index("s") * _NC + lax.axis_index("c")
    group_base = wid * (_ROWS_PER_W // _GROUP)

    def chunk_step(c, carry):
        g0 = group_base + c * _GROUPS_PER_CHUNK
        pltpu.sync_copy(idx_hbm.at[pl.ds(g0, _GROUPS_PER_CHUNK)], idx_v)
        copies = []
        for j in range(_GROUPS_PER_CHUNK):
            copies.append(
                pltpu.async_copy(
                    table_hbm.at[idx_v.at[j]],
                    rows_v.at[pl.ds(j * _GROUP, _GROUP)],
                    sem,
                )
            )
        for cp in copies:
            cp.wait()
        pltpu.sync_copy(rows_v, out_hbm.at[pl.ds(g0 * _GROUP, _CHUNK)])
        return carry

    lax.fori_loop(0, _CHUNKS_PER_W, chunk_step, 0)


@functools.lru_cache(maxsize=None)
def _make_sc_gather():
    @functools.partial(
        pl.kernel,
        out_type=jax.ShapeDtypeStruct((_BFH, _DP), jnp.float32),
        mesh=plsc.VectorSubcoreMesh(core_axis_name="c", subcore_axis_name="s"),
        scratch_types=[
            pltpu.VMEM((_GROUPS_PER_CHUNK, _GROUP), jnp.int32),
            pltpu.VMEM((_CHUNK, _DP), jnp.float32),
            pltpu.SemaphoreType.DMA,
        ],
        compiler_params=pltpu.CompilerParams(use_tc_tiling_on_sc=False),
    )
    def _sc_gather(table_hbm, idx_hbm, out_hbm, idx_v, rows_v, sem):
        _sc_gather_body(table_hbm, idx_hbm, out_hbm, idx_v, rows_v, sem)

    return _sc_gather


_MLP_BLK = 1024
_EMBWH = _FH * _DP


def _mlp_body(xn_ref, emb0_ref, emb1_ref, w1a_ref, w1b0_ref, w1b1_ref,
              b1_ref, w2_ref, b2_ref, w3_ref, b3_ref, out_ref):
    e0 = emb0_ref[...].reshape(_MLP_BLK, _EMBWH)
    e1 = emb1_ref[...].reshape(_MLP_BLK, _EMBWH)
    h = jnp.dot(xn_ref[...], w1a_ref[...], preferred_element_type=jnp.float32)
    h = h + jnp.dot(e0, w1b0_ref[...], preferred_element_type=jnp.float32)
    h = h + jnp.dot(e1, w1b1_ref[...], preferred_element_type=jnp.float32)
    h = jnp.maximum(h + b1_ref[...], 0.0)
    h = jnp.maximum(
        jnp.dot(h, w2_ref[...], preferred_element_type=jnp.float32)
        + b2_ref[...], 0.0)
    out_ref[...] = (
        jnp.dot(h, w3_ref[...], preferred_element_type=jnp.float32)
        + b3_ref[...])


def _tc_mlp(x_num, emb0, emb1, W1a, W1b0, W1b1, b1, W2, b2, W3, b3):
    grid = (_B // _MLP_BLK,)
    return pl.pallas_call(
        _mlp_body,
        grid=grid,
        in_specs=[
            pl.BlockSpec((_MLP_BLK, _NUM), lambda i: (i, 0)),
            pl.BlockSpec((_MLP_BLK * _FH, _DP), lambda i: (i, 0)),
            pl.BlockSpec((_MLP_BLK * _FH, _DP), lambda i: (i, 0)),
            pl.BlockSpec((_NUM, 512), lambda i: (0, 0)),
            pl.BlockSpec((_EMBWH, 512), lambda i: (0, 0)),
            pl.BlockSpec((_EMBWH, 512), lambda i: (0, 0)),
            pl.BlockSpec((1, 512), lambda i: (0, 0)),
            pl.BlockSpec((512, 256), lambda i: (0, 0)),
            pl.BlockSpec((1, 256), lambda i: (0, 0)),
            pl.BlockSpec((256, 1), lambda i: (0, 0)),
            pl.BlockSpec((1, 1), lambda i: (0, 0)),
        ],
        out_specs=pl.BlockSpec((_MLP_BLK, 1), lambda i: (i, 0)),
        out_shape=jax.ShapeDtypeStruct((_B, 1), jnp.float32),
    )(x_num, emb0, emb1, W1a, W1b0, W1b1, b1, W2, b2, W3, b3)


def kernel(x_num, x_cat, tables, W1, b1, W2, b2, W3, b3):
    tables_t = jnp.transpose(tables, (0, 2, 1))
    gather = _make_sc_gather()
    embs = []
    for g in range(2):
        table_pad = _tc_pad(tables_t[g * _FH:(g + 1) * _FH])
        idx = (x_cat[:, g * _FH:(g + 1) * _FH].astype(jnp.int32)
               + jnp.arange(_FH, dtype=jnp.int32)[None, :] * _V)
        idx2 = idx.reshape(_BFH // _GROUP, _GROUP)
        embs.append(gather(table_pad, idx2))
    w1e = W1[_NUM:].reshape(_F, _D, 512)
    w1e = jnp.pad(w1e, ((0, 0), (0, _DP - _D), (0, 0)))
    w1b = [w1e[g * _FH:(g + 1) * _FH].reshape(_EMBWH, 512) for g in range(2)]
    out = _tc_mlp(
        x_num, embs[0], embs[1],
        W1[:_NUM], w1b[0], w1b[1],
        b1.reshape(1, 512), W2, b2.reshape(1, 256), W3, b3.reshape(1, 1))
    return out


# speedup vs baseline: 1.0004x; 1.0004x over previous
import functools

import jax
import jax.numpy as jnp
from jax import lax
from jax.experimental import pallas as pl
from jax.experimental.pallas import tpu as pltpu
from jax.experimental.pallas import tpu_sc as plsc

_B = 16384
_F = 26
_FH = 13
_V = 100000
_D = 50
_DP = 128
_NUM = 13
_BFH = _B * _FH

_NC = 2
_NS = 16
_NW = _NC * _NS

_ROWS_PER_W = _BFH // _NW
_GROUP = 128
_GROUPS_PER_CHUNK = 4
_CHUNK = _GROUP * _GROUPS_PER_CHUNK
_CHUNKS_PER_W = _ROWS_PER_W // _CHUNK

_VCHUNKS = [(k * 6400, 6400) for k in range(15)] + [(96000, 4000)]
_VCMAX = 6400


def _in_copy(t_ref, xin_v, in_sems, f, b):
    return pltpu.make_async_copy(t_ref.at[f], xin_v.at[b], in_sems.at[b])


def _pad_body(t_ref, out_ref, xin_v, xt_v, in_sems, out_sems):
    f = pl.program_id(0)
    nf = pl.num_programs(0)

    @pl.when(f == 0)
    def _():
        _in_copy(t_ref, xin_v, in_sems, 0, 0).start()

    @pl.when(f + 1 < nf)
    def _():
        _in_copy(t_ref, xin_v, in_sems, f + 1, (f + 1) % 2).start()

    _in_copy(t_ref, xin_v, in_sems, f, f % 2).wait()

    x = xin_v.at[f % 2]
    for j, (v0, vn) in enumerate(_VCHUNKS):
        xc = x[:, pl.ds(v0, vn)]
        xt = jnp.swapaxes(xc, 0, 1)
        z = jnp.zeros((vn, _DP - _D), dtype=jnp.float32)
        if j >= 2:
            v0p, vnp = _VCHUNKS[j - 2]
            pltpu.make_async_copy(
                xt_v.at[j % 2, pl.ds(0, vnp)],
                out_ref.at[pl.ds(f * _V + v0p, vnp)], out_sems.at[j % 2]).wait()
        xt_v[j % 2, pl.ds(0, vn)] = jnp.concatenate([xt, z], axis=1)
        pltpu.make_async_copy(
            xt_v.at[j % 2, pl.ds(0, vn)],
            out_ref.at[pl.ds(f * _V + v0, vn)], out_sems.at[j % 2]).start()
    for j in (len(_VCHUNKS) - 2, len(_VCHUNKS) - 1):
        v0, vn = _VCHUNKS[j]
        pltpu.make_async_copy(
            xt_v.at[j % 2, pl.ds(0, vn)],
            out_ref.at[pl.ds(f * _V + v0, vn)], out_sems.at[j % 2]).wait()


def _tc_pad(tables_t_half):
    return pl.pallas_call(
        _pad_body,
        grid=(_FH,),
        in_specs=[pl.BlockSpec(memory_space=pltpu.HBM)],
        out_specs=pl.BlockSpec(memory_space=pltpu.HBM),
        out_shape=jax.ShapeDtypeStruct((_FH * _V, _DP), jnp.float32),
        scratch_shapes=[
            pltpu.VMEM((2, _D, _V), jnp.float32),
            pltpu.VMEM((2, _VCMAX, _DP), jnp.float32),
            pltpu.SemaphoreType.DMA((2,)),
            pltpu.SemaphoreType.DMA((2,)),
        ],
    )(tables_t_half)


def _sc_gather_body(table_hbm, idx_hbm, out_hbm, idx_v, rows_v, sem):
    wid = lax.axis_index("s") * _NC + lax.axis_index("c")
    group_base = wid * (_ROWS_PER_W // _GROUP)

    def chunk_step(c, carry):
        g0 = group_base + c * _GROUPS_PER_CHUNK
        pltpu.sync_copy(idx_hbm.at[pl.ds(g0, _GROUPS_PER_CHUNK)], idx_v)
        copies = []
        for j in range(_GROUPS_PER_CHUNK):
            copies.append(
                pltpu.async_copy(
                    table_hbm.at[idx_v.at[j]],
                    rows_v.at[pl.ds(j * _GROUP, _GROUP)],
                    sem,
                )
            )
        for cp in copies:
            cp.wait()
        pltpu.sync_copy(rows_v, out_hbm.at[pl.ds(g0 * _GROUP, _CHUNK)])
        return carry

    lax.fori_loop(0, _CHUNKS_PER_W, chunk_step, 0)


@functools.lru_cache(maxsize=None)
def _make_sc_gather():
    @functools.partial(
        pl.kernel,
        out_type=jax.ShapeDtypeStruct((_BFH, _DP), jnp.float32),
        mesh=plsc.VectorSubcoreMesh(core_axis_name="c", subcore_axis_name="s"),
        scratch_types=[
            pltpu.VMEM((_GROUPS_PER_CHUNK, _GROUP), jnp.int32),
            pltpu.VMEM((_CHUNK, _DP), jnp.float32),
            pltpu.SemaphoreType.DMA,
        ],
        compiler_params=pltpu.CompilerParams(use_tc_tiling_on_sc=False),
    )
    def _sc_gather(table_hbm, idx_hbm, out_hbm, idx_v, rows_v, sem):
        _sc_gather_body(table_hbm, idx_hbm, out_hbm, idx_v, rows_v, sem)

    return _sc_gather


_MLP_BLK = 1024
_EMBWH = _FH * _DP


def _mlp_body(xn_ref, emb0_ref, emb1_ref, w1a_ref, w1b0_ref, w1b1_ref,
              b1_ref, w2_ref, b2_ref, w3_ref, b3_ref, out_ref):
    e0 = emb0_ref[...].reshape(_MLP_BLK, _EMBWH)
    e1 = emb1_ref[...].reshape(_MLP_BLK, _EMBWH)
    h = jnp.dot(xn_ref[...], w1a_ref[...], preferred_element_type=jnp.float32)
    h = h + jnp.dot(e0, w1b0_ref[...], preferred_element_type=jnp.float32)
    h = h + jnp.dot(e1, w1b1_ref[...], preferred_element_type=jnp.float32)
    h = jnp.maximum(h + b1_ref[...], 0.0)
    h = jnp.maximum(
        jnp.dot(h, w2_ref[...], preferred_element_type=jnp.float32)
        + b2_ref[...], 0.0)
    out_ref[...] = (
        jnp.dot(h, w3_ref[...], preferred_element_type=jnp.float32)
        + b3_ref[...])


def _tc_mlp(x_num, emb0, emb1, W1a, W1b0, W1b1, b1, W2, b2, W3, b3):
    grid = (_B // _MLP_BLK,)
    return pl.pallas_call(
        _mlp_body,
        grid=grid,
        in_specs=[
            pl.BlockSpec((_MLP_BLK, _NUM), lambda i: (i, 0)),
            pl.BlockSpec((_MLP_BLK * _FH, _DP), lambda i: (i, 0)),
            pl.BlockSpec((_MLP_BLK * _FH, _DP), lambda i: (i, 0)),
            pl.BlockSpec((_NUM, 512), lambda i: (0, 0)),
            pl.BlockSpec((_EMBWH, 512), lambda i: (0, 0)),
            pl.BlockSpec((_EMBWH, 512), lambda i: (0, 0)),
            pl.BlockSpec((1, 512), lambda i: (0, 0)),
            pl.BlockSpec((512, 256), lambda i: (0, 0)),
            pl.BlockSpec((1, 256), lambda i: (0, 0)),
            pl.BlockSpec((256, 1), lambda i: (0, 0)),
            pl.BlockSpec((1, 1), lambda i: (0, 0)),
        ],
        out_specs=pl.BlockSpec((_MLP_BLK, 1), lambda i: (i, 0)),
        out_shape=jax.ShapeDtypeStruct((_B, 1), jnp.float32),
    )(x_num, emb0, emb1, W1a, W1b0, W1b1, b1, W2, b2, W3, b3)


def kernel(x_num, x_cat, tables, W1, b1, W2, b2, W3, b3):
    gather = _make_sc_gather()
    embs = []
    for g in range(2):
        half_t = jnp.transpose(tables[g * _FH:(g + 1) * _FH], (0, 2, 1))
        table_pad = _tc_pad(half_t)
        idx = (x_cat[:, g * _FH:(g + 1) * _FH].astype(jnp.int32)
               + jnp.arange(_FH, dtype=jnp.int32)[None, :] * _V)
        idx2 = idx.reshape(_BFH // _GROUP, _GROUP)
        embs.append(gather(table_pad, idx2))
    w1e = W1[_NUM:].reshape(_F, _D, 512)
    w1e = jnp.pad(w1e, ((0, 0), (0, _DP - _D), (0, 0)))
    w1b = [w1e[g * _FH:(g + 1) * _FH].reshape(_EMBWH, 512) for g in range(2)]
    out = _tc_mlp(
        x_num, embs[0], embs[1],
        W1[:_NUM], w1b[0], w1b[1],
        b1.reshape(1, 512), W2, b2.reshape(1, 256), W3, b3.reshape(1, 1))
    return out


# speedup vs baseline: 1.3538x; 1.3533x over previous
import functools

import jax
import jax.numpy as jnp
from jax import lax
from jax.experimental import pallas as pl
from jax.experimental.pallas import tpu as pltpu
from jax.experimental.pallas import tpu_sc as plsc

_B = 16384
_F = 26
_FH = 13
_V = 100000
_D = 50
_DP = 128
_NUM = 13
_BFH = _B * _FH

_NC = 2
_NS = 16
_NW = _NC * _NS

_ROWS_PER_W = _BFH // _NW
_GROUP = 128
_GROUPS_PER_CHUNK = 4
_CHUNK = _GROUP * _GROUPS_PER_CHUNK
_CHUNKS_PER_W = _ROWS_PER_W // _CHUNK

_VCHUNKS = [(k * 6400, 6400) for k in range(15)] + [(96000, 4000)]
_VCMAX = 6400


def _in_copy(t_ref, xin_v, in_sems, f, b):
    return pltpu.make_async_copy(t_ref.at[f], xin_v.at[b], in_sems.at[b])


def _pad_body(g, t_ref, out_ref, xin_v, xt_v, in_sems, out_sems):
    f = pl.program_id(0)
    nf = pl.num_programs(0)
    f0 = g * _FH

    @pl.when(f == 0)
    def _():
        _in_copy(t_ref, xin_v, in_sems, f0, 0).start()

    @pl.when(f + 1 < nf)
    def _():
        _in_copy(t_ref, xin_v, in_sems, f0 + f + 1, (f + 1) % 2).start()

    _in_copy(t_ref, xin_v, in_sems, f0 + f, f % 2).wait()

    x = xin_v.at[f % 2]
    for j, (v0, vn) in enumerate(_VCHUNKS):
        xc = x[:, pl.ds(v0, vn)]
        xt = jnp.swapaxes(xc, 0, 1)
        z = jnp.zeros((vn, _DP - _D), dtype=jnp.float32)
        if j >= 2:
            v0p, vnp = _VCHUNKS[j - 2]
            pltpu.make_async_copy(
                xt_v.at[j % 2, pl.ds(0, vnp)],
                out_ref.at[pl.ds(f * _V + v0p, vnp)], out_sems.at[j % 2]).wait()
        xt_v[j % 2, pl.ds(0, vn)] = jnp.concatenate([xt, z], axis=1)
        pltpu.make_async_copy(
            xt_v.at[j % 2, pl.ds(0, vn)],
            out_ref.at[pl.ds(f * _V + v0, vn)], out_sems.at[j % 2]).start()
    for j in (len(_VCHUNKS) - 2, len(_VCHUNKS) - 1):
        v0, vn = _VCHUNKS[j]
        pltpu.make_async_copy(
            xt_v.at[j % 2, pl.ds(0, vn)],
            out_ref.at[pl.ds(f * _V + v0, vn)], out_sems.at[j % 2]).wait()


def _tc_pad(tables_t, g):
    return pl.pallas_call(
        functools.partial(_pad_body, g),
        grid=(_FH,),
        in_specs=[pl.BlockSpec(memory_space=pltpu.HBM)],
        out_specs=pl.BlockSpec(memory_space=pltpu.HBM),
        out_shape=jax.ShapeDtypeStruct((_FH * _V, _DP), jnp.float32),
        scratch_shapes=[
            pltpu.VMEM((2, _D, _V), jnp.float32),
            pltpu.VMEM((2, _VCMAX, _DP), jnp.float32),
            pltpu.SemaphoreType.DMA((2,)),
            pltpu.SemaphoreType.DMA((2,)),
        ],
    )(tables_t)


def _sc_gather_body(table_hbm, idx_hbm, out_hbm, idx_v, rows_v, sem):
    wid = lax.axis_index("s") * _NC + lax.axis_index("c")
    group_base = wid * (_ROWS_PER_W // _GROUP)

    def chunk_step(c, carry):
        g0 = group_base + c * _GROUPS_PER_CHUNK
        pltpu.sync_copy(idx_hbm.at[pl.ds(g0, _GROUPS_PER_CHUNK)], idx_v)
        copies = []
        for j in range(_GROUPS_PER_CHUNK):
            copies.append(
                pltpu.async_copy(
                    table_hbm.at[idx_v.at[j]],
                    rows_v.at[pl.ds(j * _GROUP, _GROUP)],
                    sem,
                )
            )
        for cp in copies:
            cp.wait()
        pltpu.sync_copy(rows_v, out_hbm.at[pl.ds(g0 * _GROUP, _CHUNK)])
        return carry

    lax.fori_loop(0, _CHUNKS_PER_W, chunk_step, 0)


@functools.lru_cache(maxsize=None)
def _make_sc_gather():
    @functools.partial(
        pl.kernel,
        out_type=jax.ShapeDtypeStruct((_BFH, _DP), jnp.float32),
        mesh=plsc.VectorSubcoreMesh(core_axis_name="c", subcore_axis_name="s"),
        scratch_types=[
            pltpu.VMEM((_GROUPS_PER_CHUNK, _GROUP), jnp.int32),
            pltpu.VMEM((_CHUNK, _DP), jnp.float32),
            pltpu.SemaphoreType.DMA,
        ],
        compiler_params=pltpu.CompilerParams(use_tc_tiling_on_sc=False),
    )
    def _sc_gather(table_hbm, idx_hbm, out_hbm, idx_v, rows_v, sem):
        _sc_gather_body(table_hbm, idx_hbm, out_hbm, idx_v, rows_v, sem)

    return _sc_gather


_MLP_BLK = 1024
_EMBWH = _FH * _DP


def _mlp_body(xn_ref, emb0_ref, emb1_ref, w1a_ref, w1b0_ref, w1b1_ref,
              b1_ref, w2_ref, b2_ref, w3_ref, b3_ref, out_ref):
    e0 = emb0_ref[...].reshape(_MLP_BLK, _EMBWH)
    e1 = emb1_ref[...].reshape(_MLP_BLK, _EMBWH)
    h = jnp.dot(xn_ref[...], w1a_ref[...], preferred_element_type=jnp.float32)
    h = h + jnp.dot(e0, w1b0_ref[...], preferred_element_type=jnp.float32)
    h = h + jnp.dot(e1, w1b1_ref[...], preferred_element_type=jnp.float32)
    h = jnp.maximum(h + b1_ref[...], 0.0)
    h = jnp.maximum(
        jnp.dot(h, w2_ref[...], preferred_element_type=jnp.float32)
        + b2_ref[...], 0.0)
    out_ref[...] = (
        jnp.dot(h, w3_ref[...], preferred_element_type=jnp.float32)
        + b3_ref[...])


def _tc_mlp(x_num, emb0, emb1, W1a, W1b0, W1b1, b1, W2, b2, W3, b3):
    grid = (_B // _MLP_BLK,)
    return pl.pallas_call(
        _mlp_body,
        grid=grid,
        in_specs=[
            pl.BlockSpec((_MLP_BLK, _NUM), lambda i: (i, 0)),
            pl.BlockSpec((_MLP_BLK * _FH, _DP), lambda i: (i, 0)),
            pl.BlockSpec((_MLP_BLK * _FH, _DP), lambda i: (i, 0)),
            pl.BlockSpec((_NUM, 512), lambda i: (0, 0)),
            pl.BlockSpec((_EMBWH, 512), lambda i: (0, 0)),
            pl.BlockSpec((_EMBWH, 512), lambda i: (0, 0)),
            pl.BlockSpec((1, 512), lambda i: (0, 0)),
            pl.BlockSpec((512, 256), lambda i: (0, 0)),
            pl.BlockSpec((1, 256), lambda i: (0, 0)),
            pl.BlockSpec((256, 1), lambda i: (0, 0)),
            pl.BlockSpec((1, 1), lambda i: (0, 0)),
        ],
        out_specs=pl.BlockSpec((_MLP_BLK, 1), lambda i: (i, 0)),
        out_shape=jax.ShapeDtypeStruct((_B, 1), jnp.float32),
    )(x_num, emb0, emb1, W1a, W1b0, W1b1, b1, W2, b2, W3, b3)


def kernel(x_num, x_cat, tables, W1, b1, W2, b2, W3, b3):
    tables_t = jnp.transpose(tables, (0, 2, 1))
    gather = _make_sc_gather()
    embs = []
    for g in range(2):
        table_pad = _tc_pad(tables_t, g)
        idx = (x_cat[:, g * _FH:(g + 1) * _FH].astype(jnp.int32)
               + jnp.arange(_FH, dtype=jnp.int32)[None, :] * _V)
        idx2 = idx.reshape(_BFH // _GROUP, _GROUP)
        embs.append(gather(table_pad, idx2))
    w1e = W1[_NUM:].reshape(_F, _D, 512)
    w1e = jnp.pad(w1e, ((0, 0), (0, _DP - _D), (0, 0)))
    w1b = [w1e[g * _FH:(g + 1) * _FH].reshape(_EMBWH, 512) for g in range(2)]
    out = _tc_mlp(
        x_num, embs[0], embs[1],
        W1[:_NUM], w1b[0], w1b[1],
        b1.reshape(1, 512), W2, b2.reshape(1, 256), W3, b3.reshape(1, 1))
    return out
